# baseline (speedup 1.0000x reference)
import functools

import jax
import jax.numpy as jnp
from jax import lax
from jax.experimental import pallas as pl
from jax.experimental.pallas import tpu as pltpu
from jax.experimental.pallas import tpu_sc as plsc

B = 8
N = 10000
F_IN = 16
F_OUT = 128
E = 80000
N_HEADS = 4
N_ACT = 6
FEAT = N * F_OUT

SC_NC = 2
SC_NS = 16
SC_NW = SC_NC * SC_NS
SC_CHUNK = 125
SC_ROUNDS = E // (SC_NW * SC_CHUNK)
SC_RPT = 640
SC_NPAD = SC_NS * SC_RPT

S1_ROWS = 1000
S2_K = 25600


def _sc_segment_sum(x0, src3, dst3, zrows):
  mesh = plsc.VectorSubcoreMesh(core_axis_name="c", subcore_axis_name="s")

  @functools.partial(
      pl.kernel,
      out_type=jax.ShapeDtypeStruct((SC_NC, SC_NS, SC_RPT, F_IN),
                                    jnp.float32),
      mesh=mesh,
      scratch_types=[
          pltpu.VMEM_SHARED((SC_NPAD, F_IN), jnp.float32),
          pltpu.VMEM((SC_ROUNDS, SC_CHUNK), jnp.int32),
          pltpu.VMEM((SC_ROUNDS, SC_CHUNK), jnp.int32),
          pltpu.VMEM((SC_CHUNK, F_IN), jnp.float32),
          pltpu.SemaphoreType.DMA,
      ],
      compiler_params=pltpu.CompilerParams(use_tc_tiling_on_sc=False),
  )
  def k(x0_hbm, src_hbm, dst_hbm, z_hbm, out_hbm, s_sh, src_v, dst_v, rows_v,
        sem):
    c = lax.axis_index("c")
    s = lax.axis_index("s")
    wid = s * SC_NC + c
    pltpu.sync_copy(z_hbm, s_sh.at[pl.ds(s * SC_RPT, SC_RPT)])
    pltpu.sync_copy(src_hbm.at[wid], src_v)
    pltpu.sync_copy(dst_hbm.at[wid], dst_v)
    plsc.subcore_barrier()
    for r in range(SC_ROUNDS):
      pltpu.async_copy(x0_hbm.at[src_v.at[r]], rows_v, sem).wait()
      pltpu.sync_copy(rows_v, s_sh.at[dst_v.at[r]], add=True)
    plsc.subcore_barrier()
    pltpu.sync_copy(s_sh.at[pl.ds(s * SC_RPT, SC_RPT)], out_hbm.at[c, s])

  return k(x0, src3, dst3, zrows)


def _stage1(xf, s2, w1, b1, w2r, b2r, conv_root, conv_bias):
  nblocks = (B * N) // S1_ROWS
  nb0 = N // S1_ROWS

  def body(x_ref, s2_ref, w1_ref, b1_ref, w2r_ref, b2r_ref, cr_ref, cb_ref,
           out_ref, ws_ref):
    i = pl.program_id(0)

    @pl.when(i == 0)
    def _():
      w = b2r_ref[...]
      for kk in range(64):
        hk = jnp.maximum(w1_ref[0, kk] + b1_ref[0, kk], 0.0)
        w = w + hk * w2r_ref[kk]
      ws_ref[...] = w

    base = (
        jnp.dot(x_ref[...], cr_ref[...], preferred_element_type=jnp.float32)
        + cb_ref[...])

    @pl.when(i < nb0)
    def _():
      ssum = s2_ref[0] + s2_ref[1]
      agg = jnp.dot(ssum, ws_ref[...], preferred_element_type=jnp.float32)
      out_ref[...] = jnp.maximum(base + agg, 0.0)

    @pl.when(i >= nb0)
    def _():
      out_ref[...] = jnp.maximum(base, 0.0)

  return pl.pallas_call(
      body,
      grid=(nblocks,),
      in_specs=[
          pl.BlockSpec((S1_ROWS, F_IN), lambda i: (i, 0)),
          pl.BlockSpec((SC_NC, S1_ROWS, F_IN),
                       lambda i: (0, jnp.minimum(i, nb0 - 1), 0)),
          pl.BlockSpec(memory_space=pltpu.SMEM),
          pl.BlockSpec(memory_space=pltpu.SMEM),
          pl.BlockSpec((64, F_IN, F_OUT), lambda i: (0, 0, 0)),
          pl.BlockSpec((F_IN, F_OUT), lambda i: (0, 0)),
          pl.BlockSpec((F_IN, F_OUT), lambda i: (0, 0)),
          pl.BlockSpec((1, F_OUT), lambda i: (0, 0)),
      ],
      out_specs=pl.BlockSpec((S1_ROWS, F_OUT), lambda i: (i, 0)),
      out_shape=jax.ShapeDtypeStruct((B * N, F_OUT), jnp.float32),
      scratch_shapes=[pltpu.VMEM((F_IN, F_OUT), jnp.float32)],
  )(xf, s2, w1, b1, w2r, b2r, conv_root, conv_bias)


def _stage2(hfeat, adv_w, adv_b, val1_w, val1_b, val2_w, val2_b, val3_w,
            val3_b):
  nsteps = FEAT // S2_K
  n_adv = N_HEADS * N_ACT

  def body(h_ref, aw_ref, vw_ref, ab_ref, vb1_ref, vw2_ref, vb2_ref, vw3_ref,
           vb3_ref, out_ref, acc_a, acc_v):
    i = pl.program_id(0)
    pa = jnp.dot(h_ref[...], aw_ref[...], preferred_element_type=jnp.float32)
    pv = jnp.dot(h_ref[...], vw_ref[...], preferred_element_type=jnp.float32)

    @pl.when(i == 0)
    def _():
      acc_a[...] = pa
      acc_v[...] = pv

    @pl.when(i > 0)
    def _():
      acc_a[...] += pa
      acc_v[...] += pv

    @pl.when(i == nsteps - 1)
    def _():
      adv = jnp.maximum(acc_a[...] + ab_ref[...], 0.0)
      v = jnp.maximum(acc_v[...] + vb1_ref[...], 0.0)
      v = jnp.maximum(
          jnp.dot(v, vw2_ref[...], preferred_element_type=jnp.float32)
          + vb2_ref[...], 0.0)
      v = (jnp.dot(v, vw3_ref[...], preferred_element_type=jnp.float32)
           + vb3_ref[...])
      ga = (lax.broadcasted_iota(jnp.int32, (n_adv, N_HEADS), 0) // N_ACT ==
            lax.broadcasted_iota(jnp.int32, (n_adv, N_HEADS), 1))
      gt = (lax.broadcasted_iota(jnp.int32, (N_HEADS, n_adv), 1) // N_ACT ==
            lax.broadcasted_iota(jnp.int32, (N_HEADS, n_adv), 0))
      ga = ga.astype(jnp.float32)
      gt = gt.astype(jnp.float32)
      mean = jnp.dot(adv, ga, preferred_element_type=jnp.float32) / N_ACT
      out_ref[...] = adv + jnp.dot(
          v - mean, gt, preferred_element_type=jnp.float32)

  return pl.pallas_call(
      body,
      grid=(nsteps,),
      in_specs=[
          pl.BlockSpec((B, S2_K), lambda i: (0, i)),
          pl.BlockSpec((S2_K, n_adv), lambda i: (i, 0)),
          pl.BlockSpec((S2_K, 64), lambda i: (i, 0)),
          pl.BlockSpec((1, n_adv), lambda i: (0, 0)),
          pl.BlockSpec((1, 64), lambda i: (0, 0)),
          pl.BlockSpec((64, 64), lambda i: (0, 0)),
          pl.BlockSpec((1, 64), lambda i: (0, 0)),
          pl.BlockSpec((64, N_HEADS), lambda i: (0, 0)),
          pl.BlockSpec((1, N_HEADS), lambda i: (0, 0)),
      ],
      out_specs=pl.BlockSpec((B, n_adv), lambda i: (0, 0)),
      out_shape=jax.ShapeDtypeStruct((B, n_adv), jnp.float32),
      scratch_shapes=[
          pltpu.VMEM((B, n_adv), jnp.float32),
          pltpu.VMEM((B, 64), jnp.float32),
      ],
  )(hfeat, adv_w, val1_w, adv_b, val1_b, val2_w, val2_b, val3_w, val3_b)


def kernel(x, edge_index, nn1_w1, nn1_b1, nn1_w2, nn1_b2, conv_root,
           conv_bias, adv_w, adv_b, val1_w, val1_b, val2_w, val2_b, val3_w,
           val3_b):
  xf = x.reshape(B * N, F_IN)
  x0 = x[0]
  src3 = edge_index[0].reshape(SC_NW, SC_ROUNDS, SC_CHUNK)
  dst3 = edge_index[1].reshape(SC_NW, SC_ROUNDS, SC_CHUNK)
  zrows = jnp.zeros((SC_RPT, F_IN), jnp.float32)

  s2 = _sc_segment_sum(x0, src3, dst3, zrows)
  s2 = s2.reshape(SC_NC, SC_NPAD, F_IN)[:, :N]

  hfeat = _stage1(
      xf, s2,
      nn1_w1.reshape(1, 64), nn1_b1.reshape(1, 64),
      nn1_w2.reshape(64, F_IN, F_OUT), nn1_b2.reshape(F_IN, F_OUT),
      conv_root, conv_bias.reshape(1, F_OUT))

  q24 = _stage2(
      hfeat.reshape(B, FEAT), adv_w, adv_b.reshape(1, N_HEADS * N_ACT),
      val1_w, val1_b.reshape(1, 64), val2_w, val2_b.reshape(1, 64),
      val3_w, val3_b.reshape(1, N_HEADS))

  return q24.reshape(B, N_HEADS, N_ACT)
